# Initial kernel scaffold; baseline (speedup 1.0000x reference)
import jax
import jax.numpy as jnp
from jax.experimental import pallas as pl


def kernel(x, edge_index, W0, b0, W1, b1, W2, b2, Wc, bc):
    raise NotImplementedError("write your pallas kernel here")


# speedup vs baseline: 58.4413x; 58.4413x over previous
import functools

import jax
import jax.numpy as jnp
from jax import lax
from jax.experimental import pallas as pl
from jax.experimental.pallas import tpu as pltpu
from jax.experimental.pallas import tpu_sc as plsc

N = 10000
E = 160000
D = 256
H = 256
C = 10
K = 10
ALPHA = 0.1

NW = 16
E_PER = E // NW
N_PAD = 10240
S_PER = N_PAD // NW
L = 16


def _rsqrt16(x):
    i = plsc.bitcast(x, jnp.int32)
    i = jnp.int32(0x5F3759DF) - jnp.right_shift(i, 1)
    y = plsc.bitcast(i, jnp.float32)
    for _ in range(3):
        y = y * (1.5 - 0.5 * x * y * y)
    return y


def _sc_propagate_body(src_hbm, dst_hbm, v0_hbm, w_hbm,
                       src_v, dst_v, we_v, v_v, acc_v, nrm_v,
                       tmp2d_v, vsl_v, wsl_v, parts_sh, bcast_sh):
    wid = lax.axis_index("s")
    ebase = wid * E_PER
    sbase = wid * S_PER

    pltpu.sync_copy(src_hbm.at[pl.ds(ebase, E_PER)], src_v)
    pltpu.sync_copy(dst_hbm.at[pl.ds(ebase, E_PER)], dst_v)
    pltpu.sync_copy(v0_hbm, v_v)

    zeros16 = jnp.zeros((L,), jnp.float32)
    ones16 = jnp.ones((L,), jnp.float32)

    def zero_acc():
        def zbody(i, c):
            acc_v[pl.ds(i * L, L)] = zeros16
            return c
        lax.fori_loop(0, N_PAD // L, zbody, 0)

    zero_acc()

    def degbody(i, c):
        dsts = dst_v[pl.ds(i * L, L)]
        plsc.addupdate_scatter(acc_v, [dsts], ones16)
        return c
    lax.fori_loop(0, E_PER // L, degbody, 0)

    pltpu.sync_copy(acc_v, parts_sh.at[wid])
    plsc.subcore_barrier()
    pltpu.sync_copy(parts_sh.at[:, pl.ds(sbase, S_PER)], tmp2d_v)

    def nrmbody(j, c):
        s = tmp2d_v[0, pl.ds(j * L, L)]
        for p in range(1, NW):
            s = s + tmp2d_v[p, pl.ds(j * L, L)]
        s = jnp.maximum(s, 1.0)
        vsl_v[pl.ds(j * L, L)] = _rsqrt16(s)
        return c
    lax.fori_loop(0, S_PER // L, nrmbody, 0)

    pltpu.sync_copy(vsl_v, bcast_sh.at[pl.ds(sbase, S_PER)])
    plsc.subcore_barrier()
    pltpu.sync_copy(bcast_sh, nrm_v)
    plsc.subcore_barrier()

    def webody(i, c):
        s_idx = src_v[pl.ds(i * L, L)]
        d_idx = dst_v[pl.ds(i * L, L)]
        ns = plsc.load_gather(nrm_v, [s_idx])
        nd = plsc.load_gather(nrm_v, [d_idx])
        we_v[pl.ds(i * L, L)] = ns * nd
        return c
    lax.fori_loop(0, E_PER // L, webody, 0)

    def winit(j, c):
        wsl_v[pl.ds(j * L, L)] = ALPHA * v_v[pl.ds(sbase + j * L, L)]
        return c
    lax.fori_loop(0, S_PER // L, winit, 0)

    for it in range(K):
        zero_acc()

        def edgebody(i, c):
            d_idx = dst_v[pl.ds(i * L, L)]
            vals = plsc.load_gather(v_v, [d_idx]) * we_v[pl.ds(i * L, L)]
            s_idx = src_v[pl.ds(i * L, L)]
            plsc.addupdate_scatter(acc_v, [s_idx], vals)
            return c
        lax.fori_loop(0, E_PER // L, edgebody, 0)

        pltpu.sync_copy(acc_v, parts_sh.at[wid])
        plsc.subcore_barrier()
        pltpu.sync_copy(parts_sh.at[:, pl.ds(sbase, S_PER)], tmp2d_v)

        cdamp = (1.0 - ALPHA) ** (it + 1)
        coef = ALPHA * cdamp if it < K - 1 else cdamp

        def combody(j, c):
            s = tmp2d_v[0, pl.ds(j * L, L)]
            for p in range(1, NW):
                s = s + tmp2d_v[p, pl.ds(j * L, L)]
            vsl_v[pl.ds(j * L, L)] = s
            wsl_v[pl.ds(j * L, L)] = wsl_v[pl.ds(j * L, L)] + coef * s
            return c
        lax.fori_loop(0, S_PER // L, combody, 0)

        pltpu.sync_copy(vsl_v, bcast_sh.at[pl.ds(sbase, S_PER)])
        plsc.subcore_barrier()
        pltpu.sync_copy(bcast_sh, v_v)
        plsc.subcore_barrier()

    pltpu.sync_copy(wsl_v, w_hbm.at[pl.ds(sbase, S_PER)])


_sc_propagate = functools.partial(
    pl.kernel,
    out_type=jax.ShapeDtypeStruct((N_PAD,), jnp.float32),
    mesh=plsc.VectorSubcoreMesh(
        core_axis_name="c", subcore_axis_name="s", num_cores=1),
    compiler_params=pltpu.CompilerParams(needs_layout_passes=False),
    scratch_types=[
        pltpu.VMEM((E_PER,), jnp.int32),
        pltpu.VMEM((E_PER,), jnp.int32),
        pltpu.VMEM((E_PER,), jnp.float32),
        pltpu.VMEM((N_PAD,), jnp.float32),
        pltpu.VMEM((N_PAD,), jnp.float32),
        pltpu.VMEM((N_PAD,), jnp.float32),
        pltpu.VMEM((NW, S_PER), jnp.float32),
        pltpu.VMEM((S_PER,), jnp.float32),
        pltpu.VMEM((S_PER,), jnp.float32),
        pltpu.VMEM_SHARED((NW, N_PAD), jnp.float32),
        pltpu.VMEM_SHARED((N_PAD,), jnp.float32),
    ],
)(_sc_propagate_body)


R = 1000
G = N // R


def _tc_mlp_readout_body(x_ref, w_ref, w0_ref, b0_ref, w1_ref, b1_ref,
                         w2_ref, b2_ref, wc_ref, bc_ref, out_ref, acc_ref):
    i = pl.program_id(0)
    h = jnp.maximum(x_ref[...] @ w0_ref[...] + b0_ref[...], 0.0)
    h = jnp.maximum(h @ w1_ref[...] + b1_ref[...], 0.0)
    h = jnp.maximum(h @ w2_ref[...] + b2_ref[...], 0.0)
    part = w_ref[0] @ h

    @pl.when(i == 0)
    def _():
        acc_ref[...] = part

    @pl.when(i > 0)
    def _():
        acc_ref[...] = acc_ref[...] + part

    @pl.when(i == G - 1)
    def _():
        out_ref[...] = acc_ref[...] @ wc_ref[...] + bc_ref[...]


def _tc_mlp_readout(x, w3, W0, b0, W1, b1, W2, b2, Wc, bc):
    return pl.pallas_call(
        _tc_mlp_readout_body,
        grid=(G,),
        in_specs=[
            pl.BlockSpec((R, D), lambda i: (i, 0)),
            pl.BlockSpec((1, 1, R), lambda i: (i, 0, 0)),
            pl.BlockSpec((D, H), lambda i: (0, 0)),
            pl.BlockSpec((1, H), lambda i: (0, 0)),
            pl.BlockSpec((H, H), lambda i: (0, 0)),
            pl.BlockSpec((1, H), lambda i: (0, 0)),
            pl.BlockSpec((H, H), lambda i: (0, 0)),
            pl.BlockSpec((1, H), lambda i: (0, 0)),
            pl.BlockSpec((H, C), lambda i: (0, 0)),
            pl.BlockSpec((1, C), lambda i: (0, 0)),
        ],
        out_specs=pl.BlockSpec((1, C), lambda i: (0, 0)),
        out_shape=jax.ShapeDtypeStruct((1, C), jnp.float32),
        scratch_shapes=[pltpu.VMEM((1, H), jnp.float32)],
        compiler_params=pltpu.CompilerParams(
            dimension_semantics=("arbitrary",)),
    )(x, w3, W0, b0, W1, b1, W2, b2, Wc, bc)


def kernel(x, edge_index, W0, b0, W1, b1, W2, b2, Wc, bc):
    src = edge_index[0]
    dst = edge_index[1]
    v0 = jnp.where(jnp.arange(N_PAD) < N, 1.0 / N, 0.0).astype(jnp.float32)
    w_full = _sc_propagate(src, dst, v0)
    w3 = w_full[:N].reshape(G, 1, R)
    return _tc_mlp_readout(x, w3,
                           W0, b0.reshape(1, H), W1, b1.reshape(1, H),
                           W2, b2.reshape(1, H), Wc, bc.reshape(1, C))


# speedup vs baseline: 112.6927x; 1.9283x over previous
import functools

import jax
import jax.numpy as jnp
from jax import lax
from jax.experimental import pallas as pl
from jax.experimental.pallas import tpu as pltpu
from jax.experimental.pallas import tpu_sc as plsc

N = 10000
E = 160000
D = 256
H = 256
C = 10
K = 10
ALPHA = 0.1

NW = 16
E_PER = E // NW
N_PAD = 10240
S_PER = N_PAD // NW
L = 16


def _rsqrt16(x):
    i = plsc.bitcast(x, jnp.int32)
    i = jnp.int32(0x5F3759DF) - jnp.right_shift(i, 1)
    y = plsc.bitcast(i, jnp.float32)
    for _ in range(3):
        y = y * (1.5 - 0.5 * x * y * y)
    return y


def _sc_propagate_body(src_hbm, dst_hbm, w_hbm,
                       src_v, dst_v, we_v, v_v, acc_v, nrm_v,
                       tmp2d_v, vsl_v, wsl_v, parts_sh, bcast_sh):
    wid = lax.axis_index("s")
    ebase = wid * E_PER
    sbase = wid * S_PER
    EC = E_PER // L
    SC_ = S_PER // L

    pltpu.sync_copy(src_hbm.at[pl.ds(ebase, E_PER)], src_v)
    pltpu.sync_copy(dst_hbm.at[pl.ds(ebase, E_PER)], dst_v)

    zeros16 = jnp.zeros((L,), jnp.float32)
    ones16 = jnp.ones((L,), jnp.float32)

    def zero_acc():
        @plsc.parallel_loop(0, N_PAD // L, unroll=8)
        def _(i):
            acc_v[pl.ds(i * L, L)] = zeros16

    def combine_parts():
        pltpu.sync_copy(acc_v, parts_sh.at[wid])
        plsc.subcore_barrier()
        pltpu.sync_copy(parts_sh.at[:, pl.ds(sbase, S_PER)], tmp2d_v)

    zero_acc()

    @plsc.parallel_loop(0, EC, unroll=5)
    def _(i):
        dsts = dst_v[pl.ds(i * L, L)]
        plsc.addupdate_scatter(acc_v, [dsts], ones16)

    combine_parts()

    @plsc.parallel_loop(0, SC_, unroll=4)
    def _(j):
        s = tmp2d_v[0, pl.ds(j * L, L)]
        for p in range(1, NW):
            s = s + tmp2d_v[p, pl.ds(j * L, L)]
        vsl_v[pl.ds(j * L, L)] = _rsqrt16(jnp.maximum(s, 1.0))

    pltpu.sync_copy(vsl_v, bcast_sh.at[pl.ds(sbase, S_PER)])
    plsc.subcore_barrier()
    pltpu.sync_copy(bcast_sh, nrm_v)

    zero_acc()

    @plsc.parallel_loop(0, EC, unroll=4)
    def _(i):
        s_idx = src_v[pl.ds(i * L, L)]
        d_idx = dst_v[pl.ds(i * L, L)]
        we = plsc.load_gather(nrm_v, [s_idx]) * plsc.load_gather(nrm_v, [d_idx])
        we_v[pl.ds(i * L, L)] = we
        plsc.addupdate_scatter(acc_v, [s_idx], we * (1.0 / N))

    combine_parts()

    cdamp = 1.0 - ALPHA
    coef = ALPHA * cdamp if K > 1 else cdamp

    @plsc.parallel_loop(0, SC_, unroll=4)
    def _(j):
        s = tmp2d_v[0, pl.ds(j * L, L)]
        for p in range(1, NW):
            s = s + tmp2d_v[p, pl.ds(j * L, L)]
        vsl_v[pl.ds(j * L, L)] = s
        wsl_v[pl.ds(j * L, L)] = ALPHA * (1.0 / N) + coef * s

    pltpu.sync_copy(vsl_v, bcast_sh.at[pl.ds(sbase, S_PER)])
    plsc.subcore_barrier()
    pltpu.sync_copy(bcast_sh, v_v)

    for it in range(1, K):
        zero_acc()

        @plsc.parallel_loop(0, EC, unroll=4)
        def _(i):
            d_idx = dst_v[pl.ds(i * L, L)]
            vals = plsc.load_gather(v_v, [d_idx]) * we_v[pl.ds(i * L, L)]
            s_idx = src_v[pl.ds(i * L, L)]
            plsc.addupdate_scatter(acc_v, [s_idx], vals)

        combine_parts()

        cdamp = (1.0 - ALPHA) ** (it + 1)
        coef = ALPHA * cdamp if it < K - 1 else cdamp

        @plsc.parallel_loop(0, SC_, unroll=4)
        def _(j):
            s = tmp2d_v[0, pl.ds(j * L, L)]
            for p in range(1, NW):
                s = s + tmp2d_v[p, pl.ds(j * L, L)]
            vsl_v[pl.ds(j * L, L)] = s
            wsl_v[pl.ds(j * L, L)] = wsl_v[pl.ds(j * L, L)] + coef * s

        if it < K - 1:
            pltpu.sync_copy(vsl_v, bcast_sh.at[pl.ds(sbase, S_PER)])
            plsc.subcore_barrier()
            pltpu.sync_copy(bcast_sh, v_v)

    pltpu.sync_copy(wsl_v, w_hbm.at[pl.ds(sbase, S_PER)])


_sc_propagate = functools.partial(
    pl.kernel,
    out_type=jax.ShapeDtypeStruct((N_PAD,), jnp.float32),
    mesh=plsc.VectorSubcoreMesh(
        core_axis_name="c", subcore_axis_name="s", num_cores=1),
    compiler_params=pltpu.CompilerParams(needs_layout_passes=False),
    scratch_types=[
        pltpu.VMEM((E_PER,), jnp.int32),
        pltpu.VMEM((E_PER,), jnp.int32),
        pltpu.VMEM((E_PER,), jnp.float32),
        pltpu.VMEM((N_PAD,), jnp.float32),
        pltpu.VMEM((N_PAD,), jnp.float32),
        pltpu.VMEM((N_PAD,), jnp.float32),
        pltpu.VMEM((NW, S_PER), jnp.float32),
        pltpu.VMEM((S_PER,), jnp.float32),
        pltpu.VMEM((S_PER,), jnp.float32),
        pltpu.VMEM_SHARED((NW, N_PAD), jnp.float32),
        pltpu.VMEM_SHARED((N_PAD,), jnp.float32),
    ],
)(_sc_propagate_body)


R = 1000
G = N // R


def _tc_mlp_readout_body(x_ref, w_ref, w0_ref, b0_ref, w1_ref, b1_ref,
                         w2_ref, b2_ref, wc_ref, bc_ref, out_ref, acc_ref):
    i = pl.program_id(0)
    h = jnp.maximum(x_ref[...] @ w0_ref[...] + b0_ref[...], 0.0)
    h = jnp.maximum(h @ w1_ref[...] + b1_ref[...], 0.0)
    h = jnp.maximum(h @ w2_ref[...] + b2_ref[...], 0.0)
    part = w_ref[0] @ h

    @pl.when(i == 0)
    def _():
        acc_ref[...] = part

    @pl.when(i > 0)
    def _():
        acc_ref[...] = acc_ref[...] + part

    @pl.when(i == G - 1)
    def _():
        out_ref[...] = acc_ref[...] @ wc_ref[...] + bc_ref[...]


def _tc_mlp_readout(x, w3, W0, b0, W1, b1, W2, b2, Wc, bc):
    return pl.pallas_call(
        _tc_mlp_readout_body,
        grid=(G,),
        in_specs=[
            pl.BlockSpec((R, D), lambda i: (i, 0)),
            pl.BlockSpec((1, 1, R), lambda i: (i, 0, 0)),
            pl.BlockSpec((D, H), lambda i: (0, 0)),
            pl.BlockSpec((1, H), lambda i: (0, 0)),
            pl.BlockSpec((H, H), lambda i: (0, 0)),
            pl.BlockSpec((1, H), lambda i: (0, 0)),
            pl.BlockSpec((H, H), lambda i: (0, 0)),
            pl.BlockSpec((1, H), lambda i: (0, 0)),
            pl.BlockSpec((H, C), lambda i: (0, 0)),
            pl.BlockSpec((1, C), lambda i: (0, 0)),
        ],
        out_specs=pl.BlockSpec((1, C), lambda i: (0, 0)),
        out_shape=jax.ShapeDtypeStruct((1, C), jnp.float32),
        scratch_shapes=[pltpu.VMEM((1, H), jnp.float32)],
        compiler_params=pltpu.CompilerParams(
            dimension_semantics=("arbitrary",)),
    )(x, w3, W0, b0, W1, b1, W2, b2, Wc, bc)


def kernel(x, edge_index, W0, b0, W1, b1, W2, b2, Wc, bc):
    src = edge_index[0]
    dst = edge_index[1]
    w_full = _sc_propagate(src, dst)
    w3 = w_full[:N].reshape(G, 1, R)
    return _tc_mlp_readout(x, w3,
                           W0, b0.reshape(1, H), W1, b1.reshape(1, H),
                           W2, b2.reshape(1, H), Wc, bc.reshape(1, C))


# speedup vs baseline: 115.9050x; 1.0285x over previous
import functools

import jax
import jax.numpy as jnp
from jax import lax
from jax.experimental import pallas as pl
from jax.experimental.pallas import tpu as pltpu
from jax.experimental.pallas import tpu_sc as plsc

N = 10000
E = 160000
D = 256
H = 256
C = 10
K = 10
ALPHA = 0.1

NW = 16
E_PER = E // NW
N_PAD = 10240
S_PER = N_PAD // NW
L = 16


def _rsqrt16(x):
    i = plsc.bitcast(x, jnp.int32)
    i = jnp.int32(0x5F3759DF) - jnp.right_shift(i, 1)
    y = plsc.bitcast(i, jnp.float32)
    for _ in range(3):
        y = y * (1.5 - 0.5 * x * y * y)
    return y


def _sc_propagate_body(ei_hbm, w_hbm,
                       src_v, dst_v, we_v, v_v, acc_v, nrm_v,
                       tmp2d_v, vsl_v, wsl_v, parts_sh, bcast_sh):
    wid = lax.axis_index("s")
    ebase = wid * E_PER
    sbase = wid * S_PER
    EC = E_PER // L
    SC_ = S_PER // L

    pltpu.sync_copy(ei_hbm.at[pl.ds(ebase, E_PER)], src_v)
    pltpu.sync_copy(ei_hbm.at[pl.ds(E + ebase, E_PER)], dst_v)

    zeros16 = jnp.zeros((L,), jnp.float32)
    ones16 = jnp.ones((L,), jnp.float32)

    def zero_acc():
        @plsc.parallel_loop(0, N_PAD // L, unroll=8)
        def _(i):
            acc_v[pl.ds(i * L, L)] = zeros16

    def combine_parts(rezero=True):
        pltpu.sync_copy(acc_v, parts_sh.at[wid])
        if rezero:
            zero_acc()
        plsc.subcore_barrier()
        pltpu.sync_copy(parts_sh.at[:, pl.ds(sbase, S_PER)], tmp2d_v)

    zero_acc()

    @plsc.parallel_loop(0, EC, unroll=5)
    def _(i):
        dsts = dst_v[pl.ds(i * L, L)]
        plsc.addupdate_scatter(acc_v, [dsts], ones16)

    combine_parts()

    @plsc.parallel_loop(0, SC_, unroll=4)
    def _(j):
        s = tmp2d_v[0, pl.ds(j * L, L)]
        for p in range(1, NW):
            s = s + tmp2d_v[p, pl.ds(j * L, L)]
        vsl_v[pl.ds(j * L, L)] = _rsqrt16(jnp.maximum(s, 1.0))

    pltpu.sync_copy(vsl_v, bcast_sh.at[pl.ds(sbase, S_PER)])
    plsc.subcore_barrier()
    pltpu.sync_copy(bcast_sh, nrm_v)

    @plsc.parallel_loop(0, EC, unroll=4)
    def _(i):
        s_idx = src_v[pl.ds(i * L, L)]
        d_idx = dst_v[pl.ds(i * L, L)]
        we = plsc.load_gather(nrm_v, [s_idx]) * plsc.load_gather(nrm_v, [d_idx])
        we_v[pl.ds(i * L, L)] = we
        plsc.addupdate_scatter(acc_v, [s_idx], we * (1.0 / N))

    combine_parts()

    cdamp = 1.0 - ALPHA
    coef = ALPHA * cdamp if K > 1 else cdamp

    @plsc.parallel_loop(0, SC_, unroll=4)
    def _(j):
        s = tmp2d_v[0, pl.ds(j * L, L)]
        for p in range(1, NW):
            s = s + tmp2d_v[p, pl.ds(j * L, L)]
        vsl_v[pl.ds(j * L, L)] = s
        wsl_v[pl.ds(j * L, L)] = ALPHA * (1.0 / N) + coef * s

    pltpu.sync_copy(vsl_v, bcast_sh.at[pl.ds(sbase, S_PER)])
    plsc.subcore_barrier()
    pltpu.sync_copy(bcast_sh, v_v)

    for it in range(1, K):
        @plsc.parallel_loop(0, EC, unroll=4)
        def _(i):
            d_idx = dst_v[pl.ds(i * L, L)]
            vals = plsc.load_gather(v_v, [d_idx]) * we_v[pl.ds(i * L, L)]
            s_idx = src_v[pl.ds(i * L, L)]
            plsc.addupdate_scatter(acc_v, [s_idx], vals)

        combine_parts(rezero=(it < K - 1))

        cdamp = (1.0 - ALPHA) ** (it + 1)
        coef = ALPHA * cdamp if it < K - 1 else cdamp

        @plsc.parallel_loop(0, SC_, unroll=4)
        def _(j):
            s = tmp2d_v[0, pl.ds(j * L, L)]
            for p in range(1, NW):
                s = s + tmp2d_v[p, pl.ds(j * L, L)]
            vsl_v[pl.ds(j * L, L)] = s
            wsl_v[pl.ds(j * L, L)] = wsl_v[pl.ds(j * L, L)] + coef * s

        if it < K - 1:
            pltpu.sync_copy(vsl_v, bcast_sh.at[pl.ds(sbase, S_PER)])
            plsc.subcore_barrier()
            pltpu.sync_copy(bcast_sh, v_v)

    pltpu.sync_copy(wsl_v, w_hbm.at[pl.ds(sbase, S_PER)])


_sc_propagate = functools.partial(
    pl.kernel,
    out_type=jax.ShapeDtypeStruct((N_PAD,), jnp.float32),
    mesh=plsc.VectorSubcoreMesh(
        core_axis_name="c", subcore_axis_name="s", num_cores=1),
    compiler_params=pltpu.CompilerParams(needs_layout_passes=False),
    scratch_types=[
        pltpu.VMEM((E_PER,), jnp.int32),
        pltpu.VMEM((E_PER,), jnp.int32),
        pltpu.VMEM((E_PER,), jnp.float32),
        pltpu.VMEM((N_PAD,), jnp.float32),
        pltpu.VMEM((N_PAD,), jnp.float32),
        pltpu.VMEM((N_PAD,), jnp.float32),
        pltpu.VMEM((NW, S_PER), jnp.float32),
        pltpu.VMEM((S_PER,), jnp.float32),
        pltpu.VMEM((S_PER,), jnp.float32),
        pltpu.VMEM_SHARED((NW, N_PAD), jnp.float32),
        pltpu.VMEM_SHARED((N_PAD,), jnp.float32),
    ],
)(_sc_propagate_body)


R = 1000
G = N // R


def _tc_mlp_readout_body(x_ref, w_ref, w0_ref, b0_ref, w1_ref, b1_ref,
                         w2_ref, b2_ref, wc_ref, bc_ref, out_ref, acc_ref):
    i = pl.program_id(0)
    h = jnp.maximum(x_ref[...] @ w0_ref[...] + b0_ref[...], 0.0)
    h = jnp.maximum(h @ w1_ref[...] + b1_ref[...], 0.0)
    h = jnp.maximum(h @ w2_ref[...] + b2_ref[...], 0.0)
    part = w_ref[0] @ h

    @pl.when(i == 0)
    def _():
        acc_ref[...] = part

    @pl.when(i > 0)
    def _():
        acc_ref[...] = acc_ref[...] + part

    @pl.when(i == G - 1)
    def _():
        out_ref[...] = acc_ref[...] @ wc_ref[...] + bc_ref[...]


def _tc_mlp_readout(x, w3, W0, b0, W1, b1, W2, b2, Wc, bc):
    return pl.pallas_call(
        _tc_mlp_readout_body,
        grid=(G,),
        in_specs=[
            pl.BlockSpec((R, D), lambda i: (i, 0)),
            pl.BlockSpec((1, 1, R), lambda i: (i, 0, 0)),
            pl.BlockSpec((D, H), lambda i: (0, 0)),
            pl.BlockSpec((1, H), lambda i: (0, 0)),
            pl.BlockSpec((H, H), lambda i: (0, 0)),
            pl.BlockSpec((1, H), lambda i: (0, 0)),
            pl.BlockSpec((H, H), lambda i: (0, 0)),
            pl.BlockSpec((1, H), lambda i: (0, 0)),
            pl.BlockSpec((H, C), lambda i: (0, 0)),
            pl.BlockSpec((1, C), lambda i: (0, 0)),
        ],
        out_specs=pl.BlockSpec((1, C), lambda i: (0, 0)),
        out_shape=jax.ShapeDtypeStruct((1, C), jnp.float32),
        scratch_shapes=[pltpu.VMEM((1, H), jnp.float32)],
        compiler_params=pltpu.CompilerParams(
            dimension_semantics=("arbitrary",)),
    )(x, w3, W0, b0, W1, b1, W2, b2, Wc, bc)


def kernel(x, edge_index, W0, b0, W1, b1, W2, b2, Wc, bc):
    w_full = _sc_propagate(edge_index.reshape(2 * E))
    w3 = w_full[:N].reshape(G, 1, R)
    return _tc_mlp_readout(x, w3,
                           W0, b0.reshape(1, H), W1, b1.reshape(1, H),
                           W2, b2.reshape(1, H), Wc, bc.reshape(1, C))


# speedup vs baseline: 119.8893x; 1.0344x over previous
import functools

import jax
import jax.numpy as jnp
from jax import lax
from jax.experimental import pallas as pl
from jax.experimental.pallas import tpu as pltpu
from jax.experimental.pallas import tpu_sc as plsc

N = 10000
E = 160000
D = 256
H = 256
C = 10
K = 10
ALPHA = 0.1

NW = 16
E_PER = E // NW
N_PAD = 10240
S_PER = N_PAD // NW
L = 16


def _rsqrt16(x):
    i = plsc.bitcast(x, jnp.int32)
    i = jnp.int32(0x5F3759DF) - jnp.right_shift(i, 1)
    y = plsc.bitcast(i, jnp.float32)
    for _ in range(3):
        y = y * (1.5 - 0.5 * x * y * y)
    return y


def _sc_propagate_body(ei_hbm, w_hbm,
                       src_v, dst_v, we_v, v_v, acc_v, nrm_v,
                       tmp2d_v, vsl_v, wsl_v, parts_sh, bcast_sh):
    wid = lax.axis_index("s")
    ebase = wid * E_PER
    sbase = wid * S_PER
    EC = E_PER // L
    SC_ = S_PER // L

    pltpu.sync_copy(ei_hbm.at[pl.ds(ebase, E_PER)], src_v)
    pltpu.sync_copy(ei_hbm.at[pl.ds(E + ebase, E_PER)], dst_v)

    zeros16 = jnp.zeros((L,), jnp.float32)
    ones16 = jnp.ones((L,), jnp.float32)

    def zero_acc():
        @plsc.parallel_loop(0, N_PAD // L, unroll=8)
        def _(i):
            acc_v[pl.ds(i * L, L)] = zeros16

    def combine_parts(rezero=True):
        pltpu.sync_copy(acc_v, parts_sh.at[wid])
        if rezero:
            zero_acc()
        plsc.subcore_barrier()
        pltpu.sync_copy(parts_sh.at[:, pl.ds(sbase, S_PER)], tmp2d_v)

    zero_acc()

    @plsc.parallel_loop(0, EC, unroll=5)
    def _(i):
        dsts = dst_v[pl.ds(i * L, L)]
        plsc.addupdate_scatter(acc_v, [dsts], ones16)

    combine_parts()

    @plsc.parallel_loop(0, SC_, unroll=4)
    def _(j):
        s = tmp2d_v[0, pl.ds(j * L, L)]
        for p in range(1, NW):
            s = s + tmp2d_v[p, pl.ds(j * L, L)]
        vsl_v[pl.ds(j * L, L)] = _rsqrt16(jnp.maximum(s, 1.0))

    pltpu.sync_copy(vsl_v, bcast_sh.at[pl.ds(sbase, S_PER)])
    plsc.subcore_barrier()
    pltpu.sync_copy(bcast_sh, nrm_v)

    @plsc.parallel_loop(0, EC, unroll=4)
    def _(i):
        s_idx = src_v[pl.ds(i * L, L)]
        d_idx = dst_v[pl.ds(i * L, L)]
        we = plsc.load_gather(nrm_v, [s_idx]) * plsc.load_gather(nrm_v, [d_idx])
        we_v[pl.ds(i * L, L)] = we
        plsc.addupdate_scatter(acc_v, [s_idx], we * (1.0 / N))

    combine_parts()

    cdamp = 1.0 - ALPHA
    coef = ALPHA * cdamp if K > 1 else cdamp

    @plsc.parallel_loop(0, SC_, unroll=4)
    def _(j):
        s = tmp2d_v[0, pl.ds(j * L, L)]
        for p in range(1, NW):
            s = s + tmp2d_v[p, pl.ds(j * L, L)]
        vsl_v[pl.ds(j * L, L)] = s
        wsl_v[pl.ds(j * L, L)] = ALPHA * (1.0 / N) + coef * s

    pltpu.sync_copy(vsl_v, bcast_sh.at[pl.ds(sbase, S_PER)])
    plsc.subcore_barrier()
    pltpu.sync_copy(bcast_sh, v_v)

    for it in range(1, K):
        @plsc.parallel_loop(0, EC, unroll=4)
        def _(i):
            d_idx = dst_v[pl.ds(i * L, L)]
            vals = plsc.load_gather(v_v, [d_idx]) * we_v[pl.ds(i * L, L)]
            s_idx = src_v[pl.ds(i * L, L)]
            plsc.addupdate_scatter(acc_v, [s_idx], vals)

        combine_parts(rezero=(it < K - 1))

        cdamp = (1.0 - ALPHA) ** (it + 1)
        coef = ALPHA * cdamp if it < K - 1 else cdamp

        @plsc.parallel_loop(0, SC_, unroll=4)
        def _(j):
            s = tmp2d_v[0, pl.ds(j * L, L)]
            for p in range(1, NW):
                s = s + tmp2d_v[p, pl.ds(j * L, L)]
            vsl_v[pl.ds(j * L, L)] = s
            wsl_v[pl.ds(j * L, L)] = wsl_v[pl.ds(j * L, L)] + coef * s

        if it < K - 1:
            pltpu.sync_copy(vsl_v, bcast_sh.at[pl.ds(sbase, S_PER)])
            plsc.subcore_barrier()
            pltpu.sync_copy(bcast_sh, v_v)

    pltpu.sync_copy(wsl_v, w_hbm.at[pl.ds(sbase, S_PER)])


_sc_propagate = functools.partial(
    pl.kernel,
    out_type=jax.ShapeDtypeStruct((N_PAD,), jnp.float32),
    mesh=plsc.VectorSubcoreMesh(
        core_axis_name="c", subcore_axis_name="s", num_cores=1),
    compiler_params=pltpu.CompilerParams(needs_layout_passes=False),
    scratch_types=[
        pltpu.VMEM((E_PER,), jnp.int32),
        pltpu.VMEM((E_PER,), jnp.int32),
        pltpu.VMEM((E_PER,), jnp.float32),
        pltpu.VMEM((N_PAD,), jnp.float32),
        pltpu.VMEM((N_PAD,), jnp.float32),
        pltpu.VMEM((N_PAD,), jnp.float32),
        pltpu.VMEM((NW, S_PER), jnp.float32),
        pltpu.VMEM((S_PER,), jnp.float32),
        pltpu.VMEM((S_PER,), jnp.float32),
        pltpu.VMEM_SHARED((NW, N_PAD), jnp.float32),
        pltpu.VMEM_SHARED((N_PAD,), jnp.float32),
    ],
)(_sc_propagate_body)


R = 1000
G = N // R


def _tc_mlp_body(x_ref, w0_ref, b0_ref, w1_ref, b1_ref,
                 w2_ref, b2_ref, h_ref):
    h = jnp.maximum(x_ref[...] @ w0_ref[...] + b0_ref[...], 0.0)
    h = jnp.maximum(h @ w1_ref[...] + b1_ref[...], 0.0)
    h_ref[...] = jnp.maximum(h @ w2_ref[...] + b2_ref[...], 0.0)


def _tc_mlp(x, W0, b0, W1, b1, W2, b2):
    return pl.pallas_call(
        _tc_mlp_body,
        grid=(G,),
        in_specs=[
            pl.BlockSpec((R, D), lambda i: (i, 0)),
            pl.BlockSpec((D, H), lambda i: (0, 0)),
            pl.BlockSpec((1, H), lambda i: (0, 0)),
            pl.BlockSpec((H, H), lambda i: (0, 0)),
            pl.BlockSpec((1, H), lambda i: (0, 0)),
            pl.BlockSpec((H, H), lambda i: (0, 0)),
            pl.BlockSpec((1, H), lambda i: (0, 0)),
        ],
        out_specs=pl.BlockSpec((R, H), lambda i: (i, 0)),
        out_shape=jax.ShapeDtypeStruct((N, H), jnp.float32),
        compiler_params=pltpu.CompilerParams(
            dimension_semantics=("arbitrary",)),
    )(x, W0, b0, W1, b1, W2, b2)


def _tc_readout_body(h_ref, w_ref, wc_ref, bc_ref, out_ref, acc_ref):
    i = pl.program_id(0)
    part = w_ref[0] @ h_ref[...]

    @pl.when(i == 0)
    def _():
        acc_ref[...] = part

    @pl.when(i > 0)
    def _():
        acc_ref[...] = acc_ref[...] + part

    @pl.when(i == G - 1)
    def _():
        out_ref[...] = acc_ref[...] @ wc_ref[...] + bc_ref[...]


def _tc_readout(h, w3, Wc, bc):
    return pl.pallas_call(
        _tc_readout_body,
        grid=(G,),
        in_specs=[
            pl.BlockSpec((R, H), lambda i: (i, 0)),
            pl.BlockSpec((1, 1, R), lambda i: (i, 0, 0)),
            pl.BlockSpec((H, C), lambda i: (0, 0)),
            pl.BlockSpec((1, C), lambda i: (0, 0)),
        ],
        out_specs=pl.BlockSpec((1, C), lambda i: (0, 0)),
        out_shape=jax.ShapeDtypeStruct((1, C), jnp.float32),
        scratch_shapes=[pltpu.VMEM((1, H), jnp.float32)],
        compiler_params=pltpu.CompilerParams(
            dimension_semantics=("arbitrary",)),
    )(h, w3, Wc, bc)


def kernel(x, edge_index, W0, b0, W1, b1, W2, b2, Wc, bc):
    w_full = _sc_propagate(edge_index.reshape(2 * E))
    h = _tc_mlp(x, W0, b0.reshape(1, H), W1, b1.reshape(1, H),
                W2, b2.reshape(1, H))
    w3 = w_full[:N].reshape(G, 1, R)
    return _tc_readout(h, w3, Wc, bc.reshape(1, C))


# speedup vs baseline: 120.4850x; 1.0050x over previous
import functools

import jax
import jax.numpy as jnp
from jax import lax
from jax.experimental import pallas as pl
from jax.experimental.pallas import tpu as pltpu
from jax.experimental.pallas import tpu_sc as plsc

N = 10000
E = 160000
D = 256
H = 256
C = 10
K = 10
ALPHA = 0.1

NW = 16
E_PER = E // NW
N_PAD = 10240
S_PER = N_PAD // NW
L = 16


def _rsqrt16(x):
    i = plsc.bitcast(x, jnp.int32)
    i = jnp.int32(0x5F3759DF) - jnp.right_shift(i, 1)
    y = plsc.bitcast(i, jnp.float32)
    for _ in range(3):
        y = y * (1.5 - 0.5 * x * y * y)
    return y


def _sc_propagate_body(ei_hbm, w_hbm,
                       src_v, dst_v, we_v, v_v, acc_v, nrm_v,
                       tmp2d_v, vsl_v, wsl_v, parts_sh, bcast_sh):
    wid = lax.axis_index("s")
    ebase = wid * E_PER
    sbase = wid * S_PER
    EC = E_PER // L
    SC_ = S_PER // L

    pltpu.sync_copy(ei_hbm.at[pl.ds(ebase, E_PER)], src_v)
    pltpu.sync_copy(ei_hbm.at[pl.ds(E + ebase, E_PER)], dst_v)

    zeros16 = jnp.zeros((L,), jnp.float32)
    ones16 = jnp.ones((L,), jnp.float32)

    def zero_acc():
        @plsc.parallel_loop(0, N_PAD // L, unroll=8)
        def _(i):
            acc_v[pl.ds(i * L, L)] = zeros16

    def combine_parts(rezero=True):
        pltpu.sync_copy(acc_v, parts_sh.at[wid])
        if rezero:
            zero_acc()
        plsc.subcore_barrier()
        pltpu.sync_copy(parts_sh.at[:, pl.ds(sbase, S_PER)], tmp2d_v)

    zero_acc()

    @plsc.parallel_loop(0, EC, unroll=5)
    def _(i):
        dsts = dst_v[pl.ds(i * L, L)]
        plsc.addupdate_scatter(acc_v, [dsts], ones16)

    combine_parts()

    @plsc.parallel_loop(0, SC_, unroll=5)
    def _(j):
        s = tmp2d_v[0, pl.ds(j * L, L)]
        for p in range(1, NW):
            s = s + tmp2d_v[p, pl.ds(j * L, L)]
        vsl_v[pl.ds(j * L, L)] = _rsqrt16(jnp.maximum(s, 1.0))

    pltpu.sync_copy(vsl_v, bcast_sh.at[pl.ds(sbase, S_PER)])
    plsc.subcore_barrier()
    pltpu.sync_copy(bcast_sh, nrm_v)

    @plsc.parallel_loop(0, EC, unroll=5)
    def _(i):
        s_idx = src_v[pl.ds(i * L, L)]
        d_idx = dst_v[pl.ds(i * L, L)]
        we = plsc.load_gather(nrm_v, [s_idx]) * plsc.load_gather(nrm_v, [d_idx])
        we_v[pl.ds(i * L, L)] = we
        plsc.addupdate_scatter(acc_v, [s_idx], we * (1.0 / N))

    combine_parts()

    cdamp = 1.0 - ALPHA
    coef = ALPHA * cdamp if K > 1 else cdamp

    @plsc.parallel_loop(0, SC_, unroll=5)
    def _(j):
        s = tmp2d_v[0, pl.ds(j * L, L)]
        for p in range(1, NW):
            s = s + tmp2d_v[p, pl.ds(j * L, L)]
        vsl_v[pl.ds(j * L, L)] = s
        wsl_v[pl.ds(j * L, L)] = ALPHA * (1.0 / N) + coef * s

    pltpu.sync_copy(vsl_v, bcast_sh.at[pl.ds(sbase, S_PER)])
    plsc.subcore_barrier()
    pltpu.sync_copy(bcast_sh, v_v)

    for it in range(1, K):
        @plsc.parallel_loop(0, EC, unroll=5)
        def _(i):
            d_idx = dst_v[pl.ds(i * L, L)]
            vals = plsc.load_gather(v_v, [d_idx]) * we_v[pl.ds(i * L, L)]
            s_idx = src_v[pl.ds(i * L, L)]
            plsc.addupdate_scatter(acc_v, [s_idx], vals)

        combine_parts(rezero=(it < K - 1))

        cdamp = (1.0 - ALPHA) ** (it + 1)
        coef = ALPHA * cdamp if it < K - 1 else cdamp

        @plsc.parallel_loop(0, SC_, unroll=5)
        def _(j):
            s = tmp2d_v[0, pl.ds(j * L, L)]
            for p in range(1, NW):
                s = s + tmp2d_v[p, pl.ds(j * L, L)]
            vsl_v[pl.ds(j * L, L)] = s
            wsl_v[pl.ds(j * L, L)] = wsl_v[pl.ds(j * L, L)] + coef * s

        if it < K - 1:
            pltpu.sync_copy(vsl_v, bcast_sh.at[pl.ds(sbase, S_PER)])
            plsc.subcore_barrier()
            pltpu.sync_copy(bcast_sh, v_v)

    pltpu.sync_copy(wsl_v, w_hbm.at[pl.ds(sbase, S_PER)])


_sc_propagate = functools.partial(
    pl.kernel,
    out_type=jax.ShapeDtypeStruct((N_PAD,), jnp.float32),
    mesh=plsc.VectorSubcoreMesh(
        core_axis_name="c", subcore_axis_name="s", num_cores=1),
    compiler_params=pltpu.CompilerParams(needs_layout_passes=False),
    scratch_types=[
        pltpu.VMEM((E_PER,), jnp.int32),
        pltpu.VMEM((E_PER,), jnp.int32),
        pltpu.VMEM((E_PER,), jnp.float32),
        pltpu.VMEM((N_PAD,), jnp.float32),
        pltpu.VMEM((N_PAD,), jnp.float32),
        pltpu.VMEM((N_PAD,), jnp.float32),
        pltpu.VMEM((NW, S_PER), jnp.float32),
        pltpu.VMEM((S_PER,), jnp.float32),
        pltpu.VMEM((S_PER,), jnp.float32),
        pltpu.VMEM_SHARED((NW, N_PAD), jnp.float32),
        pltpu.VMEM_SHARED((N_PAD,), jnp.float32),
    ],
)(_sc_propagate_body)


R = 2000
G = N // R


def _tc_mlp_body(x_ref, w0_ref, b0_ref, w1_ref, b1_ref,
                 w2_ref, b2_ref, h_ref):
    h = jnp.maximum(x_ref[...] @ w0_ref[...] + b0_ref[...], 0.0)
    h = jnp.maximum(h @ w1_ref[...] + b1_ref[...], 0.0)
    h_ref[...] = jnp.maximum(h @ w2_ref[...] + b2_ref[...], 0.0)


def _tc_mlp(x, W0, b0, W1, b1, W2, b2):
    return pl.pallas_call(
        _tc_mlp_body,
        grid=(G,),
        in_specs=[
            pl.BlockSpec((R, D), lambda i: (i, 0)),
            pl.BlockSpec((D, H), lambda i: (0, 0)),
            pl.BlockSpec((1, H), lambda i: (0, 0)),
            pl.BlockSpec((H, H), lambda i: (0, 0)),
            pl.BlockSpec((1, H), lambda i: (0, 0)),
            pl.BlockSpec((H, H), lambda i: (0, 0)),
            pl.BlockSpec((1, H), lambda i: (0, 0)),
        ],
        out_specs=pl.BlockSpec((R, H), lambda i: (i, 0)),
        out_shape=jax.ShapeDtypeStruct((N, H), jnp.float32),
        compiler_params=pltpu.CompilerParams(
            dimension_semantics=("arbitrary",)),
    )(x, W0, b0, W1, b1, W2, b2)


def _tc_readout_body(h_ref, w_ref, wc_ref, bc_ref, out_ref, acc_ref):
    i = pl.program_id(0)
    part = w_ref[0] @ h_ref[...]

    @pl.when(i == 0)
    def _():
        acc_ref[...] = part

    @pl.when(i > 0)
    def _():
        acc_ref[...] = acc_ref[...] + part

    @pl.when(i == G - 1)
    def _():
        out_ref[...] = acc_ref[...] @ wc_ref[...] + bc_ref[...]


def _tc_readout(h, w3, Wc, bc):
    return pl.pallas_call(
        _tc_readout_body,
        grid=(G,),
        in_specs=[
            pl.BlockSpec((R, H), lambda i: (i, 0)),
            pl.BlockSpec((1, 1, R), lambda i: (i, 0, 0)),
            pl.BlockSpec((H, C), lambda i: (0, 0)),
            pl.BlockSpec((1, C), lambda i: (0, 0)),
        ],
        out_specs=pl.BlockSpec((1, C), lambda i: (0, 0)),
        out_shape=jax.ShapeDtypeStruct((1, C), jnp.float32),
        scratch_shapes=[pltpu.VMEM((1, H), jnp.float32)],
        compiler_params=pltpu.CompilerParams(
            dimension_semantics=("arbitrary",)),
    )(h, w3, Wc, bc)


def kernel(x, edge_index, W0, b0, W1, b1, W2, b2, Wc, bc):
    w_full = _sc_propagate(edge_index.reshape(2 * E))
    h = _tc_mlp(x, W0, b0.reshape(1, H), W1, b1.reshape(1, H),
                W2, b2.reshape(1, H))
    w3 = w_full[:N].reshape(G, 1, R)
    return _tc_readout(h, w3, Wc, bc.reshape(1, C))


# speedup vs baseline: 123.0944x; 1.0217x over previous
import functools

import jax
import jax.numpy as jnp
from jax import lax
from jax.experimental import pallas as pl
from jax.experimental.pallas import tpu as pltpu
from jax.experimental.pallas import tpu_sc as plsc

N = 10000
E = 160000
D = 256
H = 256
C = 10
K = 10
ALPHA = 0.1

NW = 16
E_PER = E // NW
N_PAD = 10240
S_PER = N_PAD // NW
L = 16


def _rsqrt16(x):
    i = plsc.bitcast(x, jnp.int32)
    i = jnp.int32(0x5F3759DF) - jnp.right_shift(i, 1)
    y = plsc.bitcast(i, jnp.float32)
    for _ in range(3):
        y = y * (1.5 - 0.5 * x * y * y)
    return y


def _sc_propagate_body(ei_hbm, w_hbm,
                       src_v, dst_v, we_v, v_v, acc_v, nrm_v,
                       tmp2d_v, vsl_v, wsl_v, parts_sh, bcast_sh):
    wid = lax.axis_index("s")
    ebase = wid * E_PER
    sbase = wid * S_PER
    EC = E_PER // L
    SC_ = S_PER // L

    pltpu.sync_copy(ei_hbm.at[pl.ds(ebase, E_PER)], src_v)
    pltpu.sync_copy(ei_hbm.at[pl.ds(E + ebase, E_PER)], dst_v)

    zeros16 = jnp.zeros((L,), jnp.float32)
    ones16 = jnp.ones((L,), jnp.float32)

    def zero_acc():
        @plsc.parallel_loop(0, N_PAD // L, unroll=8)
        def _(i):
            acc_v[pl.ds(i * L, L)] = zeros16

    def combine_parts(rezero=True):
        pltpu.sync_copy(acc_v, parts_sh.at[wid])
        if rezero:
            zero_acc()
        plsc.subcore_barrier()
        pltpu.sync_copy(parts_sh.at[:, pl.ds(sbase, S_PER)], tmp2d_v)

    zero_acc()

    @plsc.parallel_loop(0, EC, unroll=5)
    def _(i):
        dsts = dst_v[pl.ds(i * L, L)]
        plsc.addupdate_scatter(acc_v, [dsts], ones16)

    combine_parts()

    @plsc.parallel_loop(0, SC_, unroll=5)
    def _(j):
        s = tmp2d_v[0, pl.ds(j * L, L)]
        for p in range(1, NW):
            s = s + tmp2d_v[p, pl.ds(j * L, L)]
        vsl_v[pl.ds(j * L, L)] = _rsqrt16(jnp.maximum(s, 1.0))

    pltpu.sync_copy(vsl_v, bcast_sh.at[pl.ds(sbase, S_PER)])
    plsc.subcore_barrier()
    pltpu.sync_copy(bcast_sh, nrm_v)

    @plsc.parallel_loop(0, EC, unroll=5)
    def _(i):
        s_idx = src_v[pl.ds(i * L, L)]
        d_idx = dst_v[pl.ds(i * L, L)]
        we = plsc.load_gather(nrm_v, [s_idx]) * plsc.load_gather(nrm_v, [d_idx])
        we_v[pl.ds(i * L, L)] = we
        plsc.addupdate_scatter(acc_v, [s_idx], we * (1.0 / N))

    combine_parts()

    cdamp = 1.0 - ALPHA
    coef = ALPHA * cdamp if K > 1 else cdamp

    @plsc.parallel_loop(0, SC_, unroll=5)
    def _(j):
        s = tmp2d_v[0, pl.ds(j * L, L)]
        for p in range(1, NW):
            s = s + tmp2d_v[p, pl.ds(j * L, L)]
        vsl_v[pl.ds(j * L, L)] = s
        wsl_v[pl.ds(j * L, L)] = ALPHA * (1.0 / N) + coef * s

    pltpu.sync_copy(vsl_v, bcast_sh.at[pl.ds(sbase, S_PER)])
    plsc.subcore_barrier()
    pltpu.sync_copy(bcast_sh, v_v)

    for it in range(1, K):
        @plsc.parallel_loop(0, EC, unroll=5)
        def _(i):
            d_idx = dst_v[pl.ds(i * L, L)]
            vals = plsc.load_gather(v_v, [d_idx]) * we_v[pl.ds(i * L, L)]
            s_idx = src_v[pl.ds(i * L, L)]
            plsc.addupdate_scatter(acc_v, [s_idx], vals)

        combine_parts(rezero=(it < K - 1))

        cdamp = (1.0 - ALPHA) ** (it + 1)
        coef = ALPHA * cdamp if it < K - 1 else cdamp

        @plsc.parallel_loop(0, SC_, unroll=5)
        def _(j):
            s = tmp2d_v[0, pl.ds(j * L, L)]
            for p in range(1, NW):
                s = s + tmp2d_v[p, pl.ds(j * L, L)]
            vsl_v[pl.ds(j * L, L)] = s
            wsl_v[pl.ds(j * L, L)] = wsl_v[pl.ds(j * L, L)] + coef * s

        if it < K - 1:
            pltpu.sync_copy(vsl_v, bcast_sh.at[pl.ds(sbase, S_PER)])
            plsc.subcore_barrier()
            pltpu.sync_copy(bcast_sh, v_v)

    pltpu.sync_copy(wsl_v, w_hbm.at[pl.ds(sbase, S_PER)])


_sc_propagate = functools.partial(
    pl.kernel,
    out_type=jax.ShapeDtypeStruct((N_PAD,), jnp.float32),
    mesh=plsc.VectorSubcoreMesh(
        core_axis_name="c", subcore_axis_name="s", num_cores=1),
    compiler_params=pltpu.CompilerParams(needs_layout_passes=False),
    scratch_types=[
        pltpu.VMEM((E_PER,), jnp.int32),
        pltpu.VMEM((E_PER,), jnp.int32),
        pltpu.VMEM((E_PER,), jnp.float32),
        pltpu.VMEM((N_PAD,), jnp.float32),
        pltpu.VMEM((N_PAD,), jnp.float32),
        pltpu.VMEM((N_PAD,), jnp.float32),
        pltpu.VMEM((NW, S_PER), jnp.float32),
        pltpu.VMEM((S_PER,), jnp.float32),
        pltpu.VMEM((S_PER,), jnp.float32),
        pltpu.VMEM_SHARED((NW, N_PAD), jnp.float32),
        pltpu.VMEM_SHARED((N_PAD,), jnp.float32),
    ],
)(_sc_propagate_body)


R = 2000
G = N // R


def _tc_mlp_body(x_ref, w0_ref, b0_ref, w1_ref, b1_ref,
                 w2_ref, b2_ref, h_ref):
    h = jnp.maximum(x_ref[...] @ w0_ref[...] + b0_ref[...], 0.0)
    h = jnp.maximum(h @ w1_ref[...] + b1_ref[...], 0.0)
    h = jnp.maximum(h @ w2_ref[...] + b2_ref[...], 0.0)
    h_ref[...] = h.astype(jnp.bfloat16)


def _tc_mlp(x, W0, b0, W1, b1, W2, b2):
    return pl.pallas_call(
        _tc_mlp_body,
        grid=(G,),
        in_specs=[
            pl.BlockSpec((R, D), lambda i: (i, 0)),
            pl.BlockSpec((D, H), lambda i: (0, 0)),
            pl.BlockSpec((1, H), lambda i: (0, 0)),
            pl.BlockSpec((H, H), lambda i: (0, 0)),
            pl.BlockSpec((1, H), lambda i: (0, 0)),
            pl.BlockSpec((H, H), lambda i: (0, 0)),
            pl.BlockSpec((1, H), lambda i: (0, 0)),
        ],
        out_specs=pl.BlockSpec((R, H), lambda i: (i, 0)),
        out_shape=jax.ShapeDtypeStruct((N, H), jnp.bfloat16),
        compiler_params=pltpu.CompilerParams(
            dimension_semantics=("arbitrary",)),
    )(x, W0, b0, W1, b1, W2, b2)


def _tc_readout_body(h_ref, w_ref, wc_ref, bc_ref, out_ref, acc_ref):
    i = pl.program_id(0)
    part = jax.lax.dot(w_ref[0].astype(jnp.bfloat16), h_ref[...],
                       preferred_element_type=jnp.float32)

    @pl.when(i == 0)
    def _():
        acc_ref[...] = part

    @pl.when(i > 0)
    def _():
        acc_ref[...] = acc_ref[...] + part

    @pl.when(i == G - 1)
    def _():
        out_ref[...] = acc_ref[...] @ wc_ref[...] + bc_ref[...]


def _tc_readout(h, w3, Wc, bc):
    return pl.pallas_call(
        _tc_readout_body,
        grid=(G,),
        in_specs=[
            pl.BlockSpec((R, H), lambda i: (i, 0)),
            pl.BlockSpec((1, 1, R), lambda i: (i, 0, 0)),
            pl.BlockSpec((H, C), lambda i: (0, 0)),
            pl.BlockSpec((1, C), lambda i: (0, 0)),
        ],
        out_specs=pl.BlockSpec((1, C), lambda i: (0, 0)),
        out_shape=jax.ShapeDtypeStruct((1, C), jnp.float32),
        scratch_shapes=[pltpu.VMEM((1, H), jnp.float32)],
        compiler_params=pltpu.CompilerParams(
            dimension_semantics=("arbitrary",)),
    )(h, w3, Wc, bc)


def kernel(x, edge_index, W0, b0, W1, b1, W2, b2, Wc, bc):
    w_full = _sc_propagate(edge_index.reshape(2 * E))
    h = _tc_mlp(x, W0, b0.reshape(1, H), W1, b1.reshape(1, H),
                W2, b2.reshape(1, H))
    w3 = w_full[:N].reshape(G, 1, R)
    return _tc_readout(h, w3, Wc, bc.reshape(1, C))


# speedup vs baseline: 139.8738x; 1.1363x over previous
import functools

import jax
import jax.numpy as jnp
from jax import lax
from jax.experimental import pallas as pl
from jax.experimental.pallas import tpu as pltpu
from jax.experimental.pallas import tpu_sc as plsc

N = 10000
E = 160000
D = 256
H = 256
C = 10
K = 10
ALPHA = 0.1

NW = 16
E_PER = E // NW
N_PAD = 10240
S_PER = N_PAD // NW
L = 16


def _rsqrt16(x):
    i = plsc.bitcast(x, jnp.int32)
    i = jnp.int32(0x5F3759DF) - jnp.right_shift(i, 1)
    y = plsc.bitcast(i, jnp.float32)
    for _ in range(3):
        y = y * (1.5 - 0.5 * x * y * y)
    return y


NR = 80
RPW = NR // NW


def _sc_propagate_body(ei_hbm, w_hbm,
                       src_v, dst_v, we_v, v_v, acc_v, nrm_v,
                       idx80_v, zrows_v, wsl_v, buf0_sh, buf1_sh, buf2_sh):
    wid = lax.axis_index("s")
    ebase = wid * E_PER
    rbase = wid * RPW
    EC = E_PER // L
    SC_ = S_PER // L
    bufs = [buf0_sh, buf1_sh, buf2_sh]

    pltpu.sync_copy(ei_hbm.at[pl.ds(ebase, E_PER)], src_v)
    pltpu.sync_copy(ei_hbm.at[pl.ds(E + ebase, E_PER)], dst_v)

    zeros16 = jnp.zeros((L,), jnp.float32)
    ones16 = jnp.ones((L,), jnp.float32)

    def zero_acc():
        @plsc.parallel_loop(0, NR * 8, unroll=8)
        def _(i):
            acc_v[i // 8, pl.ds((i % 8) * L, L)] = zeros16

    for j in range(RPW):
        idx80_v[pl.ds(j * L, L)] = lax.iota(jnp.int32, L) + j * L

    @plsc.parallel_loop(0, RPW * 8, unroll=8)
    def _(i):
        zrows_v[i // 8, pl.ds((i % 8) * L, L)] = zeros16

    zero_acc()
    pltpu.sync_copy(zrows_v, buf0_sh.at[pl.ds(rbase, RPW)])
    pltpu.sync_copy(zrows_v, buf1_sh.at[pl.ds(rbase, RPW)])

    @plsc.parallel_loop(0, EC, unroll=5)
    def _(i):
        d = dst_v[pl.ds(i * L, L)]
        drow = jnp.right_shift(d, 7)
        dcol = jnp.bitwise_and(d, 127)
        plsc.addupdate_scatter(acc_v, [drow, dcol], ones16)

    plsc.subcore_barrier()
    pltpu.sync_copy(acc_v, buf0_sh.at[idx80_v], add=True)
    zero_acc()
    plsc.subcore_barrier()
    pltpu.sync_copy(buf0_sh, v_v)

    @plsc.parallel_loop(0, NR * 8, unroll=4)
    def _(i):
        r = i // 8
        c = (i % 8) * L
        nrm_v[r, pl.ds(c, L)] = _rsqrt16(jnp.maximum(v_v[r, pl.ds(c, L)], 1.0))

    @plsc.parallel_loop(0, EC, unroll=4)
    def _(i):
        s = src_v[pl.ds(i * L, L)]
        d = dst_v[pl.ds(i * L, L)]
        srow = jnp.right_shift(s, 7)
        scol = jnp.bitwise_and(s, 127)
        drow = jnp.right_shift(d, 7)
        dcol = jnp.bitwise_and(d, 127)
        we = (plsc.load_gather(nrm_v, [srow, scol]) *
              plsc.load_gather(nrm_v, [drow, dcol]))
        we_v[pl.ds(i * L, L)] = we
        plsc.addupdate_scatter(acc_v, [srow, scol], we * (1.0 / N))

    pltpu.sync_copy(acc_v, buf1_sh.at[idx80_v], add=True)
    zero_acc()
    pltpu.sync_copy(zrows_v, buf2_sh.at[pl.ds(rbase, RPW)])
    plsc.subcore_barrier()
    pltpu.sync_copy(buf1_sh, v_v)

    cdamp = 1.0 - ALPHA
    coef = ALPHA * cdamp if K > 1 else cdamp

    @plsc.parallel_loop(0, SC_, unroll=5)
    def _(j):
        r = rbase + j // 8
        c = (j % 8) * L
        wsl_v[pl.ds(j * L, L)] = ALPHA * (1.0 / N) + coef * v_v[r, pl.ds(c, L)]

    for it in range(2, K + 1):
        out_sh = bufs[it % 3]
        nxt_sh = bufs[(it + 1) % 3]

        @plsc.parallel_loop(0, EC, unroll=4)
        def _(i):
            s = src_v[pl.ds(i * L, L)]
            d = dst_v[pl.ds(i * L, L)]
            srow = jnp.right_shift(s, 7)
            scol = jnp.bitwise_and(s, 127)
            drow = jnp.right_shift(d, 7)
            dcol = jnp.bitwise_and(d, 127)
            vals = plsc.load_gather(v_v, [drow, dcol]) * we_v[pl.ds(i * L, L)]
            plsc.addupdate_scatter(acc_v, [srow, scol], vals)

        pltpu.sync_copy(acc_v, out_sh.at[idx80_v], add=True)
        if it < K:
            zero_acc()
            pltpu.sync_copy(zrows_v, nxt_sh.at[pl.ds(rbase, RPW)])
        plsc.subcore_barrier()
        pltpu.sync_copy(out_sh, v_v)

        cdamp = (1.0 - ALPHA) ** it
        coef = ALPHA * cdamp if it < K else cdamp

        @plsc.parallel_loop(0, SC_, unroll=5)
        def _(j):
            r = rbase + j // 8
            c = (j % 8) * L
            wsl_v[pl.ds(j * L, L)] = (wsl_v[pl.ds(j * L, L)] +
                                      coef * v_v[r, pl.ds(c, L)])

    pltpu.sync_copy(wsl_v, w_hbm.at[pl.ds(wid * S_PER, S_PER)])


_sc_propagate = functools.partial(
    pl.kernel,
    out_type=jax.ShapeDtypeStruct((N_PAD,), jnp.float32),
    mesh=plsc.VectorSubcoreMesh(
        core_axis_name="c", subcore_axis_name="s", num_cores=1),
    compiler_params=pltpu.CompilerParams(needs_layout_passes=False),
    scratch_types=[
        pltpu.VMEM((E_PER,), jnp.int32),
        pltpu.VMEM((E_PER,), jnp.int32),
        pltpu.VMEM((E_PER,), jnp.float32),
        pltpu.VMEM((NR, 128), jnp.float32),
        pltpu.VMEM((NR, 128), jnp.float32),
        pltpu.VMEM((NR, 128), jnp.float32),
        pltpu.VMEM((NR,), jnp.int32),
        pltpu.VMEM((RPW, 128), jnp.float32),
        pltpu.VMEM((S_PER,), jnp.float32),
        pltpu.VMEM_SHARED((NR, 128), jnp.float32),
        pltpu.VMEM_SHARED((NR, 128), jnp.float32),
        pltpu.VMEM_SHARED((NR, 128), jnp.float32),
    ],
)(_sc_propagate_body)


R = 2000
G = N // R


def _tc_mlp_body(x_ref, w0_ref, b0_ref, w1_ref, b1_ref,
                 w2_ref, b2_ref, h_ref):
    h = jnp.maximum(x_ref[...] @ w0_ref[...] + b0_ref[...], 0.0)
    h = jnp.maximum(h @ w1_ref[...] + b1_ref[...], 0.0)
    h = jnp.maximum(h @ w2_ref[...] + b2_ref[...], 0.0)
    h_ref[...] = h.astype(jnp.bfloat16)


def _tc_mlp(x, W0, b0, W1, b1, W2, b2):
    return pl.pallas_call(
        _tc_mlp_body,
        grid=(G,),
        in_specs=[
            pl.BlockSpec((R, D), lambda i: (i, 0)),
            pl.BlockSpec((D, H), lambda i: (0, 0)),
            pl.BlockSpec((1, H), lambda i: (0, 0)),
            pl.BlockSpec((H, H), lambda i: (0, 0)),
            pl.BlockSpec((1, H), lambda i: (0, 0)),
            pl.BlockSpec((H, H), lambda i: (0, 0)),
            pl.BlockSpec((1, H), lambda i: (0, 0)),
        ],
        out_specs=pl.BlockSpec((R, H), lambda i: (i, 0)),
        out_shape=jax.ShapeDtypeStruct((N, H), jnp.bfloat16),
        compiler_params=pltpu.CompilerParams(
            dimension_semantics=("arbitrary",)),
    )(x, W0, b0, W1, b1, W2, b2)


def _tc_readout_body(h_ref, w_ref, wc_ref, bc_ref, out_ref, acc_ref):
    i = pl.program_id(0)
    part = jax.lax.dot(w_ref[0].astype(jnp.bfloat16), h_ref[...],
                       preferred_element_type=jnp.float32)

    @pl.when(i == 0)
    def _():
        acc_ref[...] = part

    @pl.when(i > 0)
    def _():
        acc_ref[...] = acc_ref[...] + part

    @pl.when(i == G - 1)
    def _():
        out_ref[...] = acc_ref[...] @ wc_ref[...] + bc_ref[...]


def _tc_readout(h, w3, Wc, bc):
    return pl.pallas_call(
        _tc_readout_body,
        grid=(G,),
        in_specs=[
            pl.BlockSpec((R, H), lambda i: (i, 0)),
            pl.BlockSpec((1, 1, R), lambda i: (i, 0, 0)),
            pl.BlockSpec((H, C), lambda i: (0, 0)),
            pl.BlockSpec((1, C), lambda i: (0, 0)),
        ],
        out_specs=pl.BlockSpec((1, C), lambda i: (0, 0)),
        out_shape=jax.ShapeDtypeStruct((1, C), jnp.float32),
        scratch_shapes=[pltpu.VMEM((1, H), jnp.float32)],
        compiler_params=pltpu.CompilerParams(
            dimension_semantics=("arbitrary",)),
    )(h, w3, Wc, bc)


def kernel(x, edge_index, W0, b0, W1, b1, W2, b2, Wc, bc):
    w_full = _sc_propagate(edge_index.reshape(2 * E))
    h = _tc_mlp(x, W0, b0.reshape(1, H), W1, b1.reshape(1, H),
                W2, b2.reshape(1, H))
    w3 = w_full[:N].reshape(G, 1, R)
    return _tc_readout(h, w3, Wc, bc.reshape(1, C))


# speedup vs baseline: 143.8668x; 1.0285x over previous
import functools

import jax
import jax.numpy as jnp
from jax import lax
from jax.experimental import pallas as pl
from jax.experimental.pallas import tpu as pltpu
from jax.experimental.pallas import tpu_sc as plsc

N = 10000
E = 160000
D = 256
H = 256
C = 10
K = 10
ALPHA = 0.1

NW = 16
E_PER = E // NW
N_PAD = 10240
S_PER = N_PAD // NW
L = 16


def _rsqrt16(x):
    i = plsc.bitcast(x, jnp.int32)
    i = jnp.int32(0x5F3759DF) - jnp.right_shift(i, 1)
    y = plsc.bitcast(i, jnp.float32)
    for _ in range(3):
        y = y * (1.5 - 0.5 * x * y * y)
    return y


NR = 80
RPW = NR // NW


def _sc_propagate_body(ei_hbm, w_hbm,
                       src_v, dst_v, we_v, v_v, acc_v, nrm_v,
                       idx80_v, zrows_v, wsl_v, buf0_sh, buf1_sh, buf2_sh,
                       dma_sem, dma_sem2):
    wid = lax.axis_index("s")
    ebase = wid * E_PER
    rbase = wid * RPW
    EC = E_PER // L
    SC_ = S_PER // L
    bufs = [buf0_sh, buf1_sh, buf2_sh]

    cp_dst = pltpu.async_copy(ei_hbm.at[pl.ds(E + ebase, E_PER)], dst_v,
                              dma_sem)
    cp_src = pltpu.async_copy(ei_hbm.at[pl.ds(ebase, E_PER)], src_v, dma_sem2)

    zeros16 = jnp.zeros((L,), jnp.float32)
    ones16 = jnp.ones((L,), jnp.float32)

    def zero_acc():
        @plsc.parallel_loop(0, NR * 8, unroll=8)
        def _(i):
            acc_v[i // 8, pl.ds((i % 8) * L, L)] = zeros16

    for j in range(RPW):
        idx80_v[pl.ds(j * L, L)] = lax.iota(jnp.int32, L) + j * L

    @plsc.parallel_loop(0, RPW * 8, unroll=8)
    def _(i):
        zrows_v[i // 8, pl.ds((i % 8) * L, L)] = zeros16

    zero_acc()
    pltpu.sync_copy(zrows_v, buf0_sh.at[pl.ds(rbase, RPW)])
    pltpu.sync_copy(zrows_v, buf1_sh.at[pl.ds(rbase, RPW)])

    cp_dst.wait()

    @plsc.parallel_loop(0, EC, unroll=5)
    def _(i):
        d = dst_v[pl.ds(i * L, L)]
        drow = jnp.right_shift(d, 7)
        dcol = jnp.bitwise_and(d, 127)
        plsc.addupdate_scatter(acc_v, [drow, dcol], ones16)

    plsc.subcore_barrier()
    pltpu.sync_copy(acc_v, buf0_sh.at[idx80_v], add=True)
    zero_acc()
    plsc.subcore_barrier()
    pltpu.sync_copy(buf0_sh, v_v)

    @plsc.parallel_loop(0, NR * 8, unroll=4)
    def _(i):
        r = i // 8
        c = (i % 8) * L
        nrm_v[r, pl.ds(c, L)] = _rsqrt16(jnp.maximum(v_v[r, pl.ds(c, L)], 1.0))

    cp_src.wait()

    @plsc.parallel_loop(0, EC, unroll=4)
    def _(i):
        s = src_v[pl.ds(i * L, L)]
        d = dst_v[pl.ds(i * L, L)]
        srow = jnp.right_shift(s, 7)
        scol = jnp.bitwise_and(s, 127)
        drow = jnp.right_shift(d, 7)
        dcol = jnp.bitwise_and(d, 127)
        we = (plsc.load_gather(nrm_v, [srow, scol]) *
              plsc.load_gather(nrm_v, [drow, dcol]))
        we_v[pl.ds(i * L, L)] = we
        plsc.addupdate_scatter(acc_v, [srow, scol], we * (1.0 / N))

    pltpu.sync_copy(acc_v, buf1_sh.at[idx80_v], add=True)
    zero_acc()
    pltpu.sync_copy(zrows_v, buf2_sh.at[pl.ds(rbase, RPW)])
    plsc.subcore_barrier()
    pltpu.sync_copy(buf1_sh, v_v)

    cdamp = 1.0 - ALPHA
    coef = ALPHA * cdamp if K > 1 else cdamp

    @plsc.parallel_loop(0, SC_, unroll=5)
    def _(j):
        r = rbase + j // 8
        c = (j % 8) * L
        wsl_v[pl.ds(j * L, L)] = ALPHA * (1.0 / N) + coef * v_v[r, pl.ds(c, L)]

    for it in range(2, K + 1):
        out_sh = bufs[it % 3]
        nxt_sh = bufs[(it + 1) % 3]

        @plsc.parallel_loop(0, EC, unroll=4)
        def _(i):
            s = src_v[pl.ds(i * L, L)]
            d = dst_v[pl.ds(i * L, L)]
            srow = jnp.right_shift(s, 7)
            scol = jnp.bitwise_and(s, 127)
            drow = jnp.right_shift(d, 7)
            dcol = jnp.bitwise_and(d, 127)
            vals = plsc.load_gather(v_v, [drow, dcol]) * we_v[pl.ds(i * L, L)]
            plsc.addupdate_scatter(acc_v, [srow, scol], vals)

        pltpu.sync_copy(acc_v, out_sh.at[idx80_v], add=True)
        if it < K:
            zero_acc()
            pltpu.sync_copy(zrows_v, nxt_sh.at[pl.ds(rbase, RPW)])
        plsc.subcore_barrier()
        if it < K:
            pltpu.sync_copy(out_sh, v_v)
        else:
            pltpu.sync_copy(out_sh.at[pl.ds(rbase, RPW)],
                            v_v.at[pl.ds(rbase, RPW)])

        cdamp = (1.0 - ALPHA) ** it
        coef = ALPHA * cdamp if it < K else cdamp

        @plsc.parallel_loop(0, SC_, unroll=5)
        def _(j):
            r = rbase + j // 8
            c = (j % 8) * L
            wsl_v[pl.ds(j * L, L)] = (wsl_v[pl.ds(j * L, L)] +
                                      coef * v_v[r, pl.ds(c, L)])

    pltpu.sync_copy(wsl_v, w_hbm.at[pl.ds(wid * S_PER, S_PER)])


_sc_propagate = functools.partial(
    pl.kernel,
    out_type=jax.ShapeDtypeStruct((N_PAD,), jnp.float32),
    mesh=plsc.VectorSubcoreMesh(
        core_axis_name="c", subcore_axis_name="s", num_cores=1),
    compiler_params=pltpu.CompilerParams(needs_layout_passes=False),
    scratch_types=[
        pltpu.VMEM((E_PER,), jnp.int32),
        pltpu.VMEM((E_PER,), jnp.int32),
        pltpu.VMEM((E_PER,), jnp.float32),
        pltpu.VMEM((NR, 128), jnp.float32),
        pltpu.VMEM((NR, 128), jnp.float32),
        pltpu.VMEM((NR, 128), jnp.float32),
        pltpu.VMEM((NR,), jnp.int32),
        pltpu.VMEM((RPW, 128), jnp.float32),
        pltpu.VMEM((S_PER,), jnp.float32),
        pltpu.VMEM_SHARED((NR, 128), jnp.float32),
        pltpu.VMEM_SHARED((NR, 128), jnp.float32),
        pltpu.VMEM_SHARED((NR, 128), jnp.float32),
        pltpu.SemaphoreType.DMA,
        pltpu.SemaphoreType.DMA,
    ],
)(_sc_propagate_body)


R = 2000
G = N // R


def _tc_mlp_body(x_ref, w0_ref, b0_ref, w1_ref, b1_ref,
                 w2_ref, b2_ref, h_ref):
    h = jnp.maximum(x_ref[...] @ w0_ref[...] + b0_ref[...], 0.0)
    h = jnp.maximum(h @ w1_ref[...] + b1_ref[...], 0.0)
    h = jnp.maximum(h @ w2_ref[...] + b2_ref[...], 0.0)
    h_ref[...] = h.astype(jnp.bfloat16)


def _tc_mlp(x, W0, b0, W1, b1, W2, b2):
    return pl.pallas_call(
        _tc_mlp_body,
        grid=(G,),
        in_specs=[
            pl.BlockSpec((R, D), lambda i: (i, 0)),
            pl.BlockSpec((D, H), lambda i: (0, 0)),
            pl.BlockSpec((1, H), lambda i: (0, 0)),
            pl.BlockSpec((H, H), lambda i: (0, 0)),
            pl.BlockSpec((1, H), lambda i: (0, 0)),
            pl.BlockSpec((H, H), lambda i: (0, 0)),
            pl.BlockSpec((1, H), lambda i: (0, 0)),
        ],
        out_specs=pl.BlockSpec((R, H), lambda i: (i, 0)),
        out_shape=jax.ShapeDtypeStruct((N, H), jnp.bfloat16),
        compiler_params=pltpu.CompilerParams(
            dimension_semantics=("arbitrary",)),
    )(x, W0, b0, W1, b1, W2, b2)


def _tc_readout_body(h_ref, w_ref, wc_ref, bc_ref, out_ref, acc_ref):
    i = pl.program_id(0)
    part = jax.lax.dot(w_ref[0].astype(jnp.bfloat16), h_ref[...],
                       preferred_element_type=jnp.float32)

    @pl.when(i == 0)
    def _():
        acc_ref[...] = part

    @pl.when(i > 0)
    def _():
        acc_ref[...] = acc_ref[...] + part

    @pl.when(i == G - 1)
    def _():
        out_ref[...] = acc_ref[...] @ wc_ref[...] + bc_ref[...]


def _tc_readout(h, w3, Wc, bc):
    return pl.pallas_call(
        _tc_readout_body,
        grid=(G,),
        in_specs=[
            pl.BlockSpec((R, H), lambda i: (i, 0)),
            pl.BlockSpec((1, 1, R), lambda i: (i, 0, 0)),
            pl.BlockSpec((H, C), lambda i: (0, 0)),
            pl.BlockSpec((1, C), lambda i: (0, 0)),
        ],
        out_specs=pl.BlockSpec((1, C), lambda i: (0, 0)),
        out_shape=jax.ShapeDtypeStruct((1, C), jnp.float32),
        scratch_shapes=[pltpu.VMEM((1, H), jnp.float32)],
        compiler_params=pltpu.CompilerParams(
            dimension_semantics=("arbitrary",)),
    )(h, w3, Wc, bc)


def kernel(x, edge_index, W0, b0, W1, b1, W2, b2, Wc, bc):
    w_full = _sc_propagate(edge_index.reshape(2 * E))
    h = _tc_mlp(x, W0, b0.reshape(1, H), W1, b1.reshape(1, H),
                W2, b2.reshape(1, H))
    w3 = w_full[:N].reshape(G, 1, R)
    return _tc_readout(h, w3, Wc, bc.reshape(1, C))


# speedup vs baseline: 151.7141x; 1.0545x over previous
import functools

import jax
import jax.numpy as jnp
from jax import lax
from jax.experimental import pallas as pl
from jax.experimental.pallas import tpu as pltpu
from jax.experimental.pallas import tpu_sc as plsc

N = 10000
E = 160000
D = 256
H = 256
C = 10
K = 10
ALPHA = 0.1

NW = 16
E_PER = E // NW
N_PAD = 10240
S_PER = N_PAD // NW
L = 16


def _rsqrt16(x):
    i = plsc.bitcast(x, jnp.int32)
    i = jnp.int32(0x5F3759DF) - jnp.right_shift(i, 1)
    y = plsc.bitcast(i, jnp.float32)
    for _ in range(3):
        y = y * (1.5 - 0.5 * x * y * y)
    return y


NR = 80
RPW = NR // NW


def _sc_propagate_body(ei_hbm, w_hbm,
                       src_v, dst_v, we_v, v_v, acc_v, nrm_v,
                       idx80_v, zrows_v, wsl_v, buf0_sh, buf1_sh, buf2_sh,
                       dma_sem, dma_sem2):
    wid = lax.axis_index("s")
    ebase = wid * E_PER
    rbase = wid * RPW
    EC = E_PER // L
    SC_ = S_PER // L
    bufs = [buf0_sh, buf1_sh, buf2_sh]

    cp_dst = pltpu.async_copy(ei_hbm.at[pl.ds(E + ebase, E_PER)], dst_v,
                              dma_sem)
    cp_src = pltpu.async_copy(ei_hbm.at[pl.ds(ebase, E_PER)], src_v, dma_sem2)

    zeros16 = jnp.zeros((L,), jnp.float32)
    ones16 = jnp.ones((L,), jnp.float32)

    def zero_acc():
        @plsc.parallel_loop(0, NR * 8, unroll=8)
        def _(i):
            acc_v[i // 8, pl.ds((i % 8) * L, L)] = zeros16

    for j in range(RPW):
        idx80_v[pl.ds(j * L, L)] = lax.iota(jnp.int32, L) + j * L

    @plsc.parallel_loop(0, RPW * 8, unroll=8)
    def _(i):
        zrows_v[i // 8, pl.ds((i % 8) * L, L)] = zeros16

    zero_acc()
    pltpu.sync_copy(zrows_v, buf0_sh.at[pl.ds(rbase, RPW)])
    pltpu.sync_copy(zrows_v, buf1_sh.at[pl.ds(rbase, RPW)])

    cp_dst.wait()

    @plsc.parallel_loop(0, EC, unroll=5)
    def _(i):
        d = dst_v[pl.ds(i * L, L)]
        drow = jnp.right_shift(d, 7)
        dcol = jnp.bitwise_and(d, 127)
        plsc.addupdate_scatter(acc_v, [drow, dcol], ones16)

    plsc.subcore_barrier()
    pltpu.sync_copy(acc_v, buf0_sh.at[idx80_v], add=True)
    plsc.subcore_barrier()
    cp_deg = pltpu.async_copy(buf0_sh, v_v, dma_sem)
    zero_acc()
    cp_deg.wait()

    @plsc.parallel_loop(0, NR * 8, unroll=4)
    def _(i):
        r = i // 8
        c = (i % 8) * L
        nrm_v[r, pl.ds(c, L)] = _rsqrt16(jnp.maximum(v_v[r, pl.ds(c, L)], 1.0))

    cp_src.wait()

    @plsc.parallel_loop(0, EC, unroll=4)
    def _(i):
        s = src_v[pl.ds(i * L, L)]
        d = dst_v[pl.ds(i * L, L)]
        srow = jnp.right_shift(s, 7)
        scol = jnp.bitwise_and(s, 127)
        drow = jnp.right_shift(d, 7)
        dcol = jnp.bitwise_and(d, 127)
        we = (plsc.load_gather(nrm_v, [srow, scol]) *
              plsc.load_gather(nrm_v, [drow, dcol]))
        we_v[pl.ds(i * L, L)] = we
        plsc.addupdate_scatter(acc_v, [srow, scol], we * (1.0 / N))

    cp_add1 = pltpu.async_copy(acc_v, buf1_sh.at[idx80_v], dma_sem, add=True)
    pltpu.sync_copy(zrows_v, buf2_sh.at[pl.ds(rbase, RPW)])
    cp_add1.wait()
    plsc.subcore_barrier()
    cp_rd1 = pltpu.async_copy(buf1_sh, v_v, dma_sem2)
    zero_acc()
    cp_rd1.wait()

    cdamp = 1.0 - ALPHA
    coef = ALPHA * cdamp if K > 1 else cdamp

    @plsc.parallel_loop(0, SC_, unroll=5)
    def _(j):
        r = rbase + j // 8
        c = (j % 8) * L
        wsl_v[pl.ds(j * L, L)] = ALPHA * (1.0 / N) + coef * v_v[r, pl.ds(c, L)]

    for it in range(2, K + 1):
        out_sh = bufs[it % 3]
        nxt_sh = bufs[(it + 1) % 3]

        @plsc.parallel_loop(0, EC, unroll=4)
        def _(i):
            s = src_v[pl.ds(i * L, L)]
            d = dst_v[pl.ds(i * L, L)]
            srow = jnp.right_shift(s, 7)
            scol = jnp.bitwise_and(s, 127)
            drow = jnp.right_shift(d, 7)
            dcol = jnp.bitwise_and(d, 127)
            vals = plsc.load_gather(v_v, [drow, dcol]) * we_v[pl.ds(i * L, L)]
            plsc.addupdate_scatter(acc_v, [srow, scol], vals)

        cp_add = pltpu.async_copy(acc_v, out_sh.at[idx80_v], dma_sem,
                                  add=True)
        if it < K:
            pltpu.sync_copy(zrows_v, nxt_sh.at[pl.ds(rbase, RPW)])
        cp_add.wait()
        plsc.subcore_barrier()
        if it < K:
            cp_read = pltpu.async_copy(out_sh, v_v, dma_sem2)
            zero_acc()
            cp_read.wait()
        else:
            pltpu.sync_copy(out_sh.at[pl.ds(rbase, RPW)],
                            v_v.at[pl.ds(rbase, RPW)])

        cdamp = (1.0 - ALPHA) ** it
        coef = ALPHA * cdamp if it < K else cdamp

        @plsc.parallel_loop(0, SC_, unroll=5)
        def _(j):
            r = rbase + j // 8
            c = (j % 8) * L
            wsl_v[pl.ds(j * L, L)] = (wsl_v[pl.ds(j * L, L)] +
                                      coef * v_v[r, pl.ds(c, L)])

    pltpu.sync_copy(wsl_v, w_hbm.at[pl.ds(wid * S_PER, S_PER)])


_sc_propagate = functools.partial(
    pl.kernel,
    out_type=jax.ShapeDtypeStruct((N_PAD,), jnp.float32),
    mesh=plsc.VectorSubcoreMesh(
        core_axis_name="c", subcore_axis_name="s", num_cores=1),
    compiler_params=pltpu.CompilerParams(needs_layout_passes=False),
    scratch_types=[
        pltpu.VMEM((E_PER,), jnp.int32),
        pltpu.VMEM((E_PER,), jnp.int32),
        pltpu.VMEM((E_PER,), jnp.float32),
        pltpu.VMEM((NR, 128), jnp.float32),
        pltpu.VMEM((NR, 128), jnp.float32),
        pltpu.VMEM((NR, 128), jnp.float32),
        pltpu.VMEM((NR,), jnp.int32),
        pltpu.VMEM((RPW, 128), jnp.float32),
        pltpu.VMEM((S_PER,), jnp.float32),
        pltpu.VMEM_SHARED((NR, 128), jnp.float32),
        pltpu.VMEM_SHARED((NR, 128), jnp.float32),
        pltpu.VMEM_SHARED((NR, 128), jnp.float32),
        pltpu.SemaphoreType.DMA,
        pltpu.SemaphoreType.DMA,
    ],
)(_sc_propagate_body)


R = 2000
G = N // R


def _tc_mlp_body(x_ref, w0_ref, b0_ref, w1_ref, b1_ref,
                 w2_ref, b2_ref, h_ref):
    h = jnp.maximum(x_ref[...] @ w0_ref[...] + b0_ref[...], 0.0)
    h = jnp.maximum(h @ w1_ref[...] + b1_ref[...], 0.0)
    h = jnp.maximum(h @ w2_ref[...] + b2_ref[...], 0.0)
    h_ref[...] = h.astype(jnp.bfloat16)


def _tc_mlp(x, W0, b0, W1, b1, W2, b2):
    return pl.pallas_call(
        _tc_mlp_body,
        grid=(G,),
        in_specs=[
            pl.BlockSpec((R, D), lambda i: (i, 0)),
            pl.BlockSpec((D, H), lambda i: (0, 0)),
            pl.BlockSpec((1, H), lambda i: (0, 0)),
            pl.BlockSpec((H, H), lambda i: (0, 0)),
            pl.BlockSpec((1, H), lambda i: (0, 0)),
            pl.BlockSpec((H, H), lambda i: (0, 0)),
            pl.BlockSpec((1, H), lambda i: (0, 0)),
        ],
        out_specs=pl.BlockSpec((R, H), lambda i: (i, 0)),
        out_shape=jax.ShapeDtypeStruct((N, H), jnp.bfloat16),
        compiler_params=pltpu.CompilerParams(
            dimension_semantics=("arbitrary",)),
    )(x, W0, b0, W1, b1, W2, b2)


def _tc_readout_body(h_ref, w_ref, wc_ref, bc_ref, out_ref, acc_ref):
    i = pl.program_id(0)
    part = jax.lax.dot(w_ref[0].astype(jnp.bfloat16), h_ref[...],
                       preferred_element_type=jnp.float32)

    @pl.when(i == 0)
    def _():
        acc_ref[...] = part

    @pl.when(i > 0)
    def _():
        acc_ref[...] = acc_ref[...] + part

    @pl.when(i == G - 1)
    def _():
        out_ref[...] = acc_ref[...] @ wc_ref[...] + bc_ref[...]


def _tc_readout(h, w3, Wc, bc):
    return pl.pallas_call(
        _tc_readout_body,
        grid=(G,),
        in_specs=[
            pl.BlockSpec((R, H), lambda i: (i, 0)),
            pl.BlockSpec((1, 1, R), lambda i: (i, 0, 0)),
            pl.BlockSpec((H, C), lambda i: (0, 0)),
            pl.BlockSpec((1, C), lambda i: (0, 0)),
        ],
        out_specs=pl.BlockSpec((1, C), lambda i: (0, 0)),
        out_shape=jax.ShapeDtypeStruct((1, C), jnp.float32),
        scratch_shapes=[pltpu.VMEM((1, H), jnp.float32)],
        compiler_params=pltpu.CompilerParams(
            dimension_semantics=("arbitrary",)),
    )(h, w3, Wc, bc)


def kernel(x, edge_index, W0, b0, W1, b1, W2, b2, Wc, bc):
    w_full = _sc_propagate(edge_index.reshape(2 * E))
    h = _tc_mlp(x, W0, b0.reshape(1, H), W1, b1.reshape(1, H),
                W2, b2.reshape(1, H))
    w3 = w_full[:N].reshape(G, 1, R)
    return _tc_readout(h, w3, Wc, bc.reshape(1, C))
